# baseline (speedup 1.0000x reference)
import functools
import jax
import jax.numpy as jnp
from jax import lax
from jax.experimental import pallas as pl
from jax.experimental.pallas import tpu as pltpu
from jax.experimental.pallas import tpu_sc as plsc

N = 10000
NP = 10112
E = 320000
EP = 327680
ROWS = EP // 128
RPT = ROWS // 32
RPS = NP // 16
HID = 64
BT = 48
BB, TT = 4, 12
CW = 16


def _deg_body(dst_hbm, out_hbm, idx_v, ones_v, zbuf, deg_sh):
    c = lax.axis_index("c")
    s = lax.axis_index("s")
    wid = c * 16 + s
    z16 = jnp.zeros((16,), jnp.float32)
    o16 = jnp.ones((16,), jnp.float32)

    def zb(i, carry):
        zbuf[i, :] = z16
        return carry
    lax.fori_loop(0, RPS, zb, 0)

    def ob(i, carry):
        ones_v[i, :] = o16
        return carry
    lax.fori_loop(0, 128, ob, 0)

    pltpu.sync_copy(zbuf, deg_sh.at[pl.ds(s * RPS, RPS)])
    plsc.subcore_barrier()

    def chunk(g, carry):
        pltpu.sync_copy(dst_hbm.at[pl.ds(wid * RPT + g * 8, 8)], idx_v)
        for j in range(8):
            pltpu.sync_copy(ones_v, deg_sh.at[idx_v.at[j]], add=True)
        return carry
    lax.fori_loop(0, RPT // 8, chunk, 0)

    plsc.subcore_barrier()

    @pl.when(s == 0)
    def _():
        pltpu.sync_copy(deg_sh, out_hbm.at[c])


@functools.lru_cache(maxsize=None)
def _deg_call_cached():
    mesh = plsc.VectorSubcoreMesh(core_axis_name="c", subcore_axis_name="s")
    return pl.kernel(
        _deg_body,
        mesh=mesh,
        compiler_params=pltpu.CompilerParams(use_tc_tiling_on_sc=False),
        out_type=jax.ShapeDtypeStruct((2, NP, 16), jnp.float32),
        scratch_types=[
            pltpu.VMEM((8, 128), jnp.int32),
            pltpu.VMEM((128, 16), jnp.float32),
            pltpu.VMEM((RPS, 16), jnp.float32),
            pltpu.VMEM_SHARED((NP, 16), jnp.float32),
        ],
    )


def _deg_call(dst_m):
    return _deg_call_cached()(dst_m)


def _scat_body(src_hbm, dst_hbm, y_hbm, out_hbm,
               si_v, di_v, b0, b1, b2, b3, zbuf, s_sh, y_sh,
               sg0, sg1, sg2, sg3, ss0, ss1, ss2, ss3):
    c = lax.axis_index("c")
    s = lax.axis_index("s")
    wid = c * 16 + s
    z16 = jnp.zeros((16,), jnp.float32)

    def zb(i, carry):
        zbuf[i, :] = z16
        return carry
    lax.fori_loop(0, RPS, zb, 0)

    pltpu.sync_copy(zbuf, s_sh.at[pl.ds(s * RPS, RPS)])
    pltpu.sync_copy(y_hbm.at[pl.ds(s * RPS, RPS)], y_sh.at[pl.ds(s * RPS, RPS)])
    plsc.subcore_barrier()

    bufs = (b0, b1, b2, b3)
    sgs = (sg0, sg1, sg2, sg3)
    sss = (ss0, ss1, ss2, ss3)

    def chunk(g, carry):
        base = wid * RPT + g * 8
        pltpu.sync_copy(src_hbm.at[pl.ds(base, 8)], si_v)
        pltpu.sync_copy(dst_hbm.at[pl.ds(base, 8)], di_v)
        hg = [None] * 8
        hs = [None] * 8
        hg[0] = pltpu.async_copy(y_sh.at[si_v.at[0]], bufs[0], sgs[0])
        hg[1] = pltpu.async_copy(y_sh.at[si_v.at[1]], bufs[1], sgs[1])
        for j in range(8):
            if j >= 2:
                hs[j - 2].wait()
            if j < 6:
                hg[j + 2] = pltpu.async_copy(
                    y_sh.at[si_v.at[j + 2]], bufs[(j + 2) % 4], sgs[(j + 2) % 4])
            hg[j].wait()
            hs[j] = pltpu.async_copy(
                bufs[j % 4], s_sh.at[di_v.at[j]], sss[j % 4], add=True)
        hs[6].wait()
        hs[7].wait()
        return carry
    lax.fori_loop(0, RPT // 8, chunk, 0)

    plsc.subcore_barrier()

    @pl.when(s == 0)
    def _():
        pltpu.sync_copy(s_sh, out_hbm.at[c])


@functools.lru_cache(maxsize=None)
def _scat_call_cached():
    mesh = plsc.VectorSubcoreMesh(core_axis_name="c", subcore_axis_name="s")
    return pl.kernel(
        _scat_body,
        mesh=mesh,
        compiler_params=pltpu.CompilerParams(use_tc_tiling_on_sc=False),
        out_type=jax.ShapeDtypeStruct((2, NP, CW), jnp.float32),
        scratch_types=[
            pltpu.VMEM((8, 128), jnp.int32),
            pltpu.VMEM((8, 128), jnp.int32),
            pltpu.VMEM((128, CW), jnp.float32),
            pltpu.VMEM((128, CW), jnp.float32),
            pltpu.VMEM((128, CW), jnp.float32),
            pltpu.VMEM((128, CW), jnp.float32),
            pltpu.VMEM((RPS, CW), jnp.float32),
            pltpu.VMEM_SHARED((NP, CW), jnp.float32),
            pltpu.VMEM_SHARED((NP, CW), jnp.float32),
            pltpu.SemaphoreType.DMA,
            pltpu.SemaphoreType.DMA,
            pltpu.SemaphoreType.DMA,
            pltpu.SemaphoreType.DMA,
            pltpu.SemaphoreType.DMA,
            pltpu.SemaphoreType.DMA,
            pltpu.SemaphoreType.DMA,
            pltpu.SemaphoreType.DMA,
        ],
    )


def _scat_call(src_m, dst_m, y16):
    return _scat_call_cached()(src_m, dst_m, y16)


def _prep_body(x_ref, deg_ref, y_ref, dis_ref):
    deg = deg_ref[0, :, 0:1] + deg_ref[1, :, 0:1] + 1.0
    dis = lax.rsqrt(deg)
    zp = jnp.zeros((NP, CW - TT), jnp.float32)
    for b in range(BB):
        y_ref[b] = jnp.concatenate([x_ref[b] * dis, zp], axis=1)
    dis_ref[...] = dis


def _prep_call(x4, deg_part):
    return pl.pallas_call(
        _prep_body,
        out_shape=[
            jax.ShapeDtypeStruct((BB, NP, CW), jnp.float32),
            jax.ShapeDtypeStruct((NP, 1), jnp.float32),
        ],
    )(x4, deg_part)


def _fused_body(S_ref, x_ref, dis_ref, gcnW_ref, gcnb_row_ref, saW_ref,
                taW_ref, gcnb_col_ref, Wih_ref, Whh_ref, bih_ref, bhh_ref,
                projW_ref, projb_ref, out_ref):
    d = dis_ref[...][:, :N]
    y = x_ref[...] * d
    s = d * (S_ref[0, :, :N] + S_ref[1, :, :N] + y)
    alpha = jnp.dot(gcnW_ref[...], saW_ref[...])
    beta = jnp.dot(gcnW_ref[...], taW_ref[...])
    gamma = jnp.dot(gcnb_row_ref[...], taW_ref[...])
    l1 = alpha * s
    m1 = jnp.max(l1, axis=1, keepdims=True)
    e1 = jnp.exp(l1 - m1)
    sw = e1 / jnp.sum(e1, axis=1, keepdims=True)
    u = sw * s
    v = sw
    l2 = beta * u + gamma * v
    m2 = jnp.max(l2, axis=0, keepdims=True)
    e2 = jnp.exp(l2 - m2)
    tw = e2 / jnp.sum(e2, axis=0, keepdims=True)
    p = tw * u
    q = tw * v
    Aw = lax.dot_general(Wih_ref[...], gcnW_ref[...],
                         (((1,), (1,)), ((), ())))
    Ab = lax.dot_general(Wih_ref[...], gcnb_col_ref[...],
                         (((1,), (0,)), ((), ())))
    bi = bih_ref[...]
    bh = bhh_ref[...]
    Whh = Whh_ref[...]
    H = jnp.zeros((HID, N), jnp.float32)
    for t in range(TT):
        pt = p[t:t + 1, :]
        qt = q[t:t + 1, :]
        GI = Aw * pt + Ab * qt + bi
        GH = lax.dot_general(Whh, H,
                             (((1,), (0,)), ((), ()))) + bh
        r = jax.nn.sigmoid(GI[0:HID] + GH[0:HID])
        z = jax.nn.sigmoid(GI[HID:2 * HID] + GH[HID:2 * HID])
        nn_ = jnp.tanh(GI[2 * HID:3 * HID] + r * GH[2 * HID:3 * HID])
        H = (1.0 - z) * nn_ + z * H
    OUT = lax.dot_general(projW_ref[...], H,
                          (((0,), (0,)), ((), ()))) + projb_ref[...]
    out_ref[...] = OUT


def _fused_call(S12, x_b, dis_row, gcn_W, gcn_b_row, sa_W, ta_W,
                gcn_b_col, W_ih, W_hh, b_ih_col, b_hh_col,
                proj_W, proj_b_col):
    return pl.pallas_call(
        _fused_body,
        out_shape=jax.ShapeDtypeStruct((TT, N), jnp.float32),
    )(S12, x_b, dis_row, gcn_W, gcn_b_row, sa_W, ta_W,
      gcn_b_col, W_ih, W_hh, b_ih_col, b_hh_col, proj_W, proj_b_col)


def kernel(x, edge_index, gcn_W, gcn_b, sa_W, sa_b, ta_W, ta_b,
           W_ih, W_hh, b_ih, b_hh, proj_W, proj_b):
    x2d = x.reshape(BT, N)
    src = edge_index[0].astype(jnp.int32)
    dst = edge_index[1].astype(jnp.int32)
    pad = jnp.full((EP - E,), N, jnp.int32)
    src_m = jnp.concatenate([src, pad]).reshape(ROWS, 128)
    dst_m = jnp.concatenate([dst, pad]).reshape(ROWS, 128)
    x4 = jnp.pad(jnp.transpose(x2d.reshape(BB, TT, N), (0, 2, 1)),
                 ((0, 0), (0, NP - N), (0, 0)))

    deg_part = _deg_call(dst_m)
    y4, dis_col = _prep_call(x4, deg_part)
    dis_row = dis_col.reshape(1, NP)
    gcn_b_row = gcn_b.reshape(1, HID)
    gcn_b_col = gcn_b.reshape(HID, 1)
    b_ih_col = b_ih.reshape(3 * HID, 1)
    b_hh_col = b_hh.reshape(3 * HID, 1)
    proj_b_col = proj_b.reshape(TT, 1)
    x3 = x2d.reshape(BB, TT, N)

    outs = []
    for b in range(BB):
        S_part = _scat_call(src_m, dst_m, y4[b])
        S12 = jnp.transpose(S_part, (0, 2, 1))[:, :TT, :]
        outs.append(_fused_call(S12, x3[b], dis_row, gcn_W, gcn_b_row,
                                sa_W, ta_W, gcn_b_col, W_ih, W_hh,
                                b_ih_col, b_hh_col, proj_W, proj_b_col))
    out3 = jnp.stack(outs)
    return out3[..., None]


# speedup vs baseline: 1.0969x; 1.0969x over previous
import functools
import jax
import jax.numpy as jnp
from jax import lax
from jax.experimental import pallas as pl
from jax.experimental.pallas import tpu as pltpu
from jax.experimental.pallas import tpu_sc as plsc

N = 10000
NP = 10112
E = 320000
EP = 327680
ROWS = EP // 128
RPT = ROWS // 32
RPS = NP // 16
HID = 64
BT = 48
BB, TT = 4, 12


def _deg_body(dst_hbm, out_hbm, idx_v, ones_v, zbuf, deg_sh):
    c = lax.axis_index("c")
    s = lax.axis_index("s")
    wid = c * 16 + s
    z16 = jnp.zeros((16,), jnp.float32)
    o16 = jnp.ones((16,), jnp.float32)

    def zb(i, carry):
        zbuf[i, :] = z16
        return carry
    lax.fori_loop(0, RPS, zb, 0)

    def ob(i, carry):
        ones_v[i, :] = o16
        return carry
    lax.fori_loop(0, 128, ob, 0)

    pltpu.sync_copy(zbuf, deg_sh.at[pl.ds(s * RPS, RPS)])
    plsc.subcore_barrier()

    def chunk(g, carry):
        pltpu.sync_copy(dst_hbm.at[pl.ds(wid * RPT + g * 8, 8)], idx_v)
        for j in range(8):
            pltpu.sync_copy(ones_v, deg_sh.at[idx_v.at[j]], add=True)
        return carry
    lax.fori_loop(0, RPT // 8, chunk, 0)

    plsc.subcore_barrier()

    @pl.when(s == 0)
    def _():
        pltpu.sync_copy(deg_sh, out_hbm.at[c])


@functools.lru_cache(maxsize=None)
def _deg_call_cached():
    mesh = plsc.VectorSubcoreMesh(core_axis_name="c", subcore_axis_name="s")
    return pl.kernel(
        _deg_body,
        mesh=mesh,
        compiler_params=pltpu.CompilerParams(use_tc_tiling_on_sc=False),
        out_type=jax.ShapeDtypeStruct((2, NP, 16), jnp.float32),
        scratch_types=[
            pltpu.VMEM((8, 128), jnp.int32),
            pltpu.VMEM((128, 16), jnp.float32),
            pltpu.VMEM((RPS, 16), jnp.float32),
            pltpu.VMEM_SHARED((NP, 16), jnp.float32),
        ],
    )


def _deg_call(dst_m):
    return _deg_call_cached()(dst_m)


def _scat_body(src_hbm, dst_hbm, y_hbm, out_hbm,
               si_v, di_v, r0, r1, r2, r3, zbuf, s_sh, y_sh,
               sg0, sg1, sg2, sg3, ss0, ss1, ss2, ss3):
    c = lax.axis_index("c")
    s = lax.axis_index("s")
    wid = c * 16 + s
    z16 = jnp.zeros((16,), jnp.float32)

    def zb(i, carry):
        for k in range(3):
            zbuf[i, pl.ds(k * 16, 16)] = z16
        return carry
    lax.fori_loop(0, RPS, zb, 0)

    pltpu.sync_copy(zbuf, s_sh.at[pl.ds(s * RPS, RPS)])
    pltpu.sync_copy(y_hbm.at[pl.ds(s * RPS, RPS)], y_sh.at[pl.ds(s * RPS, RPS)])
    plsc.subcore_barrier()

    bufs = (r0, r1, r2, r3)
    sgs = (sg0, sg1, sg2, sg3)
    sss = (ss0, ss1, ss2, ss3)

    def chunk(g, carry):
        base = wid * RPT + g * 8
        pltpu.sync_copy(src_hbm.at[pl.ds(base, 8)], si_v)
        pltpu.sync_copy(dst_hbm.at[pl.ds(base, 8)], di_v)
        hg = [None] * 8
        hs = [None] * 8
        hg[0] = pltpu.async_copy(y_sh.at[si_v.at[0]], bufs[0], sgs[0])
        hg[1] = pltpu.async_copy(y_sh.at[si_v.at[1]], bufs[1], sgs[1])
        for j in range(8):
            if j >= 2:
                hs[j - 2].wait()
            if j < 6:
                hg[j + 2] = pltpu.async_copy(
                    y_sh.at[si_v.at[j + 2]], bufs[(j + 2) % 4], sgs[(j + 2) % 4])
            hg[j].wait()
            hs[j] = pltpu.async_copy(
                bufs[j % 4], s_sh.at[di_v.at[j]], sss[j % 4], add=True)
        hs[6].wait()
        hs[7].wait()
        return carry
    lax.fori_loop(0, RPT // 8, chunk, 0)

    plsc.subcore_barrier()

    @pl.when(s == 0)
    def _():
        pltpu.sync_copy(s_sh, out_hbm.at[c])


@functools.lru_cache(maxsize=None)
def _scat_call_cached():
    mesh = plsc.VectorSubcoreMesh(core_axis_name="c", subcore_axis_name="s")
    return pl.kernel(
        _scat_body,
        mesh=mesh,
        compiler_params=pltpu.CompilerParams(use_tc_tiling_on_sc=False),
        out_type=jax.ShapeDtypeStruct((2, NP, 48), jnp.float32),
        scratch_types=[
            pltpu.VMEM((8, 128), jnp.int32),
            pltpu.VMEM((8, 128), jnp.int32),
            pltpu.VMEM((128, 48), jnp.float32),
            pltpu.VMEM((128, 48), jnp.float32),
            pltpu.VMEM((128, 48), jnp.float32),
            pltpu.VMEM((128, 48), jnp.float32),
            pltpu.VMEM((RPS, 48), jnp.float32),
            pltpu.VMEM_SHARED((NP, 48), jnp.float32),
            pltpu.VMEM_SHARED((NP, 48), jnp.float32),
            pltpu.SemaphoreType.DMA,
            pltpu.SemaphoreType.DMA,
            pltpu.SemaphoreType.DMA,
            pltpu.SemaphoreType.DMA,
            pltpu.SemaphoreType.DMA,
            pltpu.SemaphoreType.DMA,
            pltpu.SemaphoreType.DMA,
            pltpu.SemaphoreType.DMA,
        ],
    )


def _scat_call(src_m, dst_m, y):
    return _scat_call_cached()(src_m, dst_m, y)


def _prep_body(x_ref, deg_ref, y_ref, dis_ref):
    deg = deg_ref[0, :, 0:1] + deg_ref[1, :, 0:1] + 1.0
    dis = lax.rsqrt(deg)
    y_ref[...] = x_ref[...] * dis
    dis_ref[...] = dis


def _prep_call(x_nT, deg_part):
    return pl.pallas_call(
        _prep_body,
        out_shape=[
            jax.ShapeDtypeStruct((NP, 48), jnp.float32),
            jax.ShapeDtypeStruct((NP, 1), jnp.float32),
        ],
    )(x_nT, deg_part)


def _fused_body(S_ref, x_ref, dis_ref, gcnW_ref, gcnb_row_ref, saW_ref,
                taW_ref, gcnb_col_ref, Wih_ref, Whh_ref, bih_ref, bhh_ref,
                projW_ref, projb_ref, out_ref):
    d = dis_ref[...][:, :N]
    y = x_ref[0] * d
    s = d * (S_ref[0, 0, :, :N] + S_ref[1, 0, :, :N] + y)
    alpha = jnp.dot(gcnW_ref[...], saW_ref[...])
    beta = jnp.dot(gcnW_ref[...], taW_ref[...])
    gamma = jnp.dot(gcnb_row_ref[...], taW_ref[...])
    l1 = alpha * s
    m1 = jnp.max(l1, axis=1, keepdims=True)
    e1 = jnp.exp(l1 - m1)
    sw = e1 / jnp.sum(e1, axis=1, keepdims=True)
    u = sw * s
    v = sw
    l2 = beta * u + gamma * v
    m2 = jnp.max(l2, axis=0, keepdims=True)
    e2 = jnp.exp(l2 - m2)
    tw = e2 / jnp.sum(e2, axis=0, keepdims=True)
    p = tw * u
    q = tw * v
    Aw = lax.dot_general(Wih_ref[...], gcnW_ref[...],
                         (((1,), (1,)), ((), ())))
    Ab = lax.dot_general(Wih_ref[...], gcnb_col_ref[...],
                         (((1,), (0,)), ((), ())))
    bi = bih_ref[...]
    bh = bhh_ref[...]
    Whh = Whh_ref[...]
    H = jnp.zeros((HID, N), jnp.float32)
    for t in range(TT):
        pt = p[t:t + 1, :]
        qt = q[t:t + 1, :]
        GI = Aw * pt + Ab * qt + bi
        GH = lax.dot_general(Whh, H,
                             (((1,), (0,)), ((), ()))) + bh
        r = jax.nn.sigmoid(GI[0:HID] + GH[0:HID])
        z = jax.nn.sigmoid(GI[HID:2 * HID] + GH[HID:2 * HID])
        nn_ = jnp.tanh(GI[2 * HID:3 * HID] + r * GH[2 * HID:3 * HID])
        H = (1.0 - z) * nn_ + z * H
    OUT = lax.dot_general(projW_ref[...], H,
                          (((0,), (0,)), ((), ()))) + projb_ref[...]
    out_ref[...] = OUT[None]


def _fused_call(S_T, x2d, dis_row, gcn_W, gcn_b_row, sa_W, ta_W,
                gcn_b_col, W_ih, W_hh, b_ih_col, b_hh_col,
                proj_W, proj_b_col):
    full = lambda shape: pl.BlockSpec(shape, lambda b: tuple(0 for _ in shape))
    return pl.pallas_call(
        _fused_body,
        grid=(BB,),
        in_specs=[
            pl.BlockSpec((2, 1, TT, NP), lambda b: (0, b, 0, 0)),
            pl.BlockSpec((1, TT, N), lambda b: (b, 0, 0)),
            full((1, NP)),
            full((1, HID)),
            full((1, HID)),
            full((HID, 1)),
            full((HID, 1)),
            full((HID, 1)),
            full((3 * HID, HID)),
            full((3 * HID, HID)),
            full((3 * HID, 1)),
            full((3 * HID, 1)),
            full((HID, TT)),
            full((TT, 1)),
        ],
        out_specs=pl.BlockSpec((1, TT, N), lambda b: (b, 0, 0)),
        out_shape=jax.ShapeDtypeStruct((BB, TT, N), jnp.float32),
    )(S_T.reshape(2, BB, TT, NP), x2d.reshape(BB, TT, N), dis_row,
      gcn_W, gcn_b_row, sa_W, ta_W,
      gcn_b_col, W_ih, W_hh, b_ih_col, b_hh_col, proj_W, proj_b_col)


def kernel(x, edge_index, gcn_W, gcn_b, sa_W, sa_b, ta_W, ta_b,
           W_ih, W_hh, b_ih, b_hh, proj_W, proj_b):
    x2d = x.reshape(BT, N)
    src = edge_index[0].astype(jnp.int32)
    dst = edge_index[1].astype(jnp.int32)
    pad = jnp.full((EP - E,), N, jnp.int32)
    src_m = jnp.concatenate([src, pad]).reshape(ROWS, 128)
    dst_m = jnp.concatenate([dst, pad]).reshape(ROWS, 128)
    x_nT = jnp.pad(x2d.T, ((0, NP - N), (0, 0)))

    deg_part = _deg_call(dst_m)
    y, dis_col = _prep_call(x_nT, deg_part)
    S_part = _scat_call(src_m, dst_m, y)

    S_T = jnp.transpose(S_part, (0, 2, 1))
    dis_row = dis_col.reshape(1, NP)
    out3 = _fused_call(S_T, x2d, dis_row, gcn_W, gcn_b.reshape(1, HID),
                       sa_W, ta_W, gcn_b.reshape(HID, 1), W_ih, W_hh,
                       b_ih.reshape(3 * HID, 1), b_hh.reshape(3 * HID, 1),
                       proj_W, proj_b.reshape(TT, 1))
    return out3[..., None]


# speedup vs baseline: 1.3115x; 1.1956x over previous
import functools
import jax
import jax.numpy as jnp
from jax import lax
from jax.experimental import pallas as pl
from jax.experimental.pallas import tpu as pltpu
from jax.experimental.pallas import tpu_sc as plsc

N = 10000
NP = 10112
E = 320000
EP = 327680
ROWS = EP // 128
RPT = ROWS // 32
RPS = NP // 16
HID = 64
BT = 48
BB, TT = 4, 12


def _deg_body(dst_hbm, out_hbm, idx_v, ones_v, zbuf, deg_sh):
    c = lax.axis_index("c")
    s = lax.axis_index("s")
    wid = c * 16 + s
    z16 = jnp.zeros((16,), jnp.float32)
    o16 = jnp.ones((16,), jnp.float32)

    def zb(i, carry):
        zbuf[i, :] = z16
        return carry
    lax.fori_loop(0, RPS, zb, 0)

    def ob(i, carry):
        ones_v[i, :] = o16
        return carry
    lax.fori_loop(0, 128, ob, 0)

    pltpu.sync_copy(zbuf, deg_sh.at[pl.ds(s * RPS, RPS)])
    plsc.subcore_barrier()

    def chunk(g, carry):
        pltpu.sync_copy(dst_hbm.at[pl.ds(wid * RPT + g * 8, 8)], idx_v)
        for j in range(8):
            pltpu.sync_copy(ones_v, deg_sh.at[idx_v.at[j]], add=True)
        return carry
    lax.fori_loop(0, RPT // 8, chunk, 0)

    plsc.subcore_barrier()

    @pl.when(s == 0)
    def _():
        pltpu.sync_copy(deg_sh, out_hbm.at[c])


@functools.lru_cache(maxsize=None)
def _deg_call_cached():
    mesh = plsc.VectorSubcoreMesh(core_axis_name="c", subcore_axis_name="s")
    return pl.kernel(
        _deg_body,
        mesh=mesh,
        compiler_params=pltpu.CompilerParams(use_tc_tiling_on_sc=False),
        out_type=jax.ShapeDtypeStruct((2, NP, 16), jnp.float32),
        scratch_types=[
            pltpu.VMEM((8, 128), jnp.int32),
            pltpu.VMEM((128, 16), jnp.float32),
            pltpu.VMEM((RPS, 16), jnp.float32),
            pltpu.VMEM_SHARED((NP, 16), jnp.float32),
        ],
    )


def _deg_call(dst_m):
    return _deg_call_cached()(dst_m)


def _scat_body(src_hbm, dst_hbm, y_hbm, out_hbm,
               si_v, di_v, rows_a, rows_b, zbuf, s_sh, y_sh, sem_a, sem_b):
    c = lax.axis_index("c")
    s = lax.axis_index("s")
    wid = c * 16 + s
    z16 = jnp.zeros((16,), jnp.float32)

    def zb(i, carry):
        for k in range(3):
            zbuf[i, pl.ds(k * 16, 16)] = z16
        return carry
    lax.fori_loop(0, RPS, zb, 0)

    pltpu.sync_copy(zbuf, s_sh.at[pl.ds(s * RPS, RPS)])
    pltpu.sync_copy(y_hbm.at[pl.ds(s * RPS, RPS)], y_sh.at[pl.ds(s * RPS, RPS)])
    plsc.subcore_barrier()

    bufs = (rows_a, rows_b)
    sems = (sem_a, sem_b)

    def chunk(g, carry):
        base = wid * RPT + g * 8
        pltpu.sync_copy(src_hbm.at[pl.ds(base, 8)], si_v)
        pltpu.sync_copy(dst_hbm.at[pl.ds(base, 8)], di_v)
        h = pltpu.async_copy(y_sh.at[si_v.at[0]], bufs[0], sems[0])
        for j in range(8):
            if j < 7:
                h_next = pltpu.async_copy(
                    y_sh.at[si_v.at[j + 1]], bufs[(j + 1) % 2], sems[(j + 1) % 2])
            h.wait()
            pltpu.sync_copy(bufs[j % 2], s_sh.at[di_v.at[j]], add=True)
            if j < 7:
                h = h_next
        return carry
    lax.fori_loop(0, RPT // 8, chunk, 0)

    plsc.subcore_barrier()

    @pl.when(s == 0)
    def _():
        pltpu.sync_copy(s_sh, out_hbm.at[c])


@functools.lru_cache(maxsize=None)
def _scat_call_cached():
    mesh = plsc.VectorSubcoreMesh(core_axis_name="c", subcore_axis_name="s")
    return pl.kernel(
        _scat_body,
        mesh=mesh,
        compiler_params=pltpu.CompilerParams(use_tc_tiling_on_sc=False),
        out_type=jax.ShapeDtypeStruct((2, NP, 48), jnp.float32),
        scratch_types=[
            pltpu.VMEM((8, 128), jnp.int32),
            pltpu.VMEM((8, 128), jnp.int32),
            pltpu.VMEM((128, 48), jnp.float32),
            pltpu.VMEM((128, 48), jnp.float32),
            pltpu.VMEM((RPS, 48), jnp.float32),
            pltpu.VMEM_SHARED((NP, 48), jnp.float32),
            pltpu.VMEM_SHARED((NP, 48), jnp.float32),
            pltpu.SemaphoreType.DMA,
            pltpu.SemaphoreType.DMA,
        ],
    )


def _scat_call(src_m, dst_m, y):
    return _scat_call_cached()(src_m, dst_m, y)


def _prep_body(x_ref, deg_ref, y_ref, dis_ref):
    deg = deg_ref[0, :, 0:1] + deg_ref[1, :, 0:1] + 1.0
    dis = lax.rsqrt(deg)
    y_ref[...] = x_ref[...] * dis
    dis_ref[...] = dis


def _prep_call(x_nT, deg_part):
    return pl.pallas_call(
        _prep_body,
        out_shape=[
            jax.ShapeDtypeStruct((NP, 48), jnp.float32),
            jax.ShapeDtypeStruct((NP, 1), jnp.float32),
        ],
    )(x_nT, deg_part)


def _fused_body(S_ref, x_ref, dis_ref, gcnW_ref, gcnb_row_ref, saW_ref,
                taW_ref, gcnb_col_ref, Wih_ref, Whh_ref, bih_ref, bhh_ref,
                projW_ref, projb_ref, out_ref):
    dcol = dis_ref[...]
    rowm = (lax.broadcasted_iota(jnp.int32, (NP, 1), 0) < N
            ).astype(jnp.float32)
    y = x_ref[...] * dcol
    s = dcol * (S_ref[0] + S_ref[1] + y)
    alpha = jnp.dot(gcnW_ref[...], saW_ref[...])
    beta = jnp.dot(gcnW_ref[...], taW_ref[...])
    gamma = jnp.dot(gcnb_row_ref[...], taW_ref[...])
    e1 = jnp.exp(alpha * s) * rowm
    sw = e1 / jnp.sum(e1, axis=0, keepdims=True)
    u = sw * s
    v = sw
    e2 = jnp.exp(beta * u + gamma * v) * rowm
    gi = lax.broadcasted_iota(jnp.int32, (BT, BT), 0) // TT
    gj = lax.broadcasted_iota(jnp.int32, (BT, BT), 1) // TT
    G = (gi == gj).astype(jnp.float32)
    D2 = jnp.dot(e2, G)
    tw = e2 / D2
    pT = jnp.transpose(tw * u)
    qT = jnp.transpose(tw * v)
    Aw = lax.dot_general(Wih_ref[...], gcnW_ref[...],
                         (((1,), (1,)), ((), ())))
    Ab = lax.dot_general(Wih_ref[...], gcnb_col_ref[...],
                         (((1,), (0,)), ((), ())))
    bi = bih_ref[...]
    bh = bhh_ref[...]
    Whh = Whh_ref[...]
    for b in range(BB):
        H = jnp.zeros((HID, N), jnp.float32)
        for t in range(TT):
            pt = pT[b * TT + t:b * TT + t + 1, :N]
            qt = qT[b * TT + t:b * TT + t + 1, :N]
            GI = Aw * pt + Ab * qt + bi
            GH = lax.dot_general(Whh, H,
                                 (((1,), (0,)), ((), ()))) + bh
            r = jax.nn.sigmoid(GI[0:HID] + GH[0:HID])
            z = jax.nn.sigmoid(GI[HID:2 * HID] + GH[HID:2 * HID])
            nn_ = jnp.tanh(GI[2 * HID:3 * HID] + r * GH[2 * HID:3 * HID])
            H = (1.0 - z) * nn_ + z * H
        OUT = lax.dot_general(projW_ref[...], H,
                              (((0,), (0,)), ((), ()))) + projb_ref[...]
        out_ref[b] = OUT


def _fused_call(S_part, x_nT, dis_col, gcn_W, gcn_b_row, sa_W, ta_W,
                gcn_b_col, W_ih, W_hh, b_ih_col, b_hh_col,
                proj_W, proj_b_col):
    return pl.pallas_call(
        _fused_body,
        compiler_params=pltpu.CompilerParams(
            vmem_limit_bytes=110 * 1024 * 1024),
        out_shape=jax.ShapeDtypeStruct((BB, TT, N), jnp.float32),
    )(S_part, x_nT, dis_col, gcn_W, gcn_b_row, sa_W, ta_W,
      gcn_b_col, W_ih, W_hh, b_ih_col, b_hh_col, proj_W, proj_b_col)


def kernel(x, edge_index, gcn_W, gcn_b, sa_W, sa_b, ta_W, ta_b,
           W_ih, W_hh, b_ih, b_hh, proj_W, proj_b):
    x2d = x.reshape(BT, N)
    src = edge_index[0].astype(jnp.int32)
    dst = edge_index[1].astype(jnp.int32)
    pad = jnp.full((EP - E,), N, jnp.int32)
    src_m = jnp.concatenate([src, pad]).reshape(ROWS, 128)
    dst_m = jnp.concatenate([dst, pad]).reshape(ROWS, 128)
    x_nT = jnp.pad(x2d.T, ((0, NP - N), (0, 0)))

    deg_part = _deg_call(dst_m)
    y, dis_col = _prep_call(x_nT, deg_part)
    S_part = _scat_call(src_m, dst_m, y)

    out3 = _fused_call(S_part, x_nT, dis_col, gcn_W, gcn_b.reshape(1, HID),
                       sa_W, ta_W, gcn_b.reshape(HID, 1), W_ih, W_hh,
                       b_ih.reshape(3 * HID, 1), b_hh.reshape(3 * HID, 1),
                       proj_W, proj_b.reshape(TT, 1))
    return out3[..., None]


# speedup vs baseline: 1.3208x; 1.0071x over previous
import functools
import jax
import jax.numpy as jnp
from jax import lax
from jax.experimental import pallas as pl
from jax.experimental.pallas import tpu as pltpu
from jax.experimental.pallas import tpu_sc as plsc

N = 10000
NP = 10112
E = 320000
EP = 327680
ROWS = EP // 128
RPT = ROWS // 32
RPS = NP // 16
HID = 64
BT = 48
BB, TT = 4, 12


def _deg_body(dst_hbm, out_hbm, idx_v, ones_v, zbuf, deg_sh):
    c = lax.axis_index("c")
    s = lax.axis_index("s")
    wid = c * 16 + s
    z16 = jnp.zeros((16,), jnp.float32)
    o16 = jnp.ones((16,), jnp.float32)

    def zb(i, carry):
        zbuf[i, :] = z16
        return carry
    lax.fori_loop(0, RPS, zb, 0)

    def ob(i, carry):
        ones_v[i, :] = o16
        return carry
    lax.fori_loop(0, 128, ob, 0)

    pltpu.sync_copy(zbuf, deg_sh.at[pl.ds(s * RPS, RPS)])
    plsc.subcore_barrier()

    def chunk(g, carry):
        pltpu.sync_copy(dst_hbm.at[pl.ds(wid * RPT + g * 8, 8)], idx_v)
        for j in range(8):
            pltpu.sync_copy(ones_v, deg_sh.at[idx_v.at[j]], add=True)
        return carry
    lax.fori_loop(0, RPT // 8, chunk, 0)

    plsc.subcore_barrier()

    @pl.when(s == 0)
    def _():
        pltpu.sync_copy(deg_sh, out_hbm.at[c])


@functools.lru_cache(maxsize=None)
def _deg_call_cached():
    mesh = plsc.VectorSubcoreMesh(core_axis_name="c", subcore_axis_name="s")
    return pl.kernel(
        _deg_body,
        mesh=mesh,
        compiler_params=pltpu.CompilerParams(use_tc_tiling_on_sc=False),
        out_type=jax.ShapeDtypeStruct((2, NP, 16), jnp.float32),
        scratch_types=[
            pltpu.VMEM((8, 128), jnp.int32),
            pltpu.VMEM((128, 16), jnp.float32),
            pltpu.VMEM((RPS, 16), jnp.float32),
            pltpu.VMEM_SHARED((NP, 16), jnp.float32),
        ],
    )


def _deg_call(dst_m):
    return _deg_call_cached()(dst_m)


def _scat_body(src_hbm, dst_hbm, y_hbm, out_hbm,
               si_v, di_v, rows_a, rows_b, zbuf, s_sh, y_sh, sem_a, sem_b):
    c = lax.axis_index("c")
    s = lax.axis_index("s")
    wid = c * 16 + s
    z16 = jnp.zeros((16,), jnp.float32)

    def zb(i, carry):
        for k in range(3):
            zbuf[i, pl.ds(k * 16, 16)] = z16
        return carry
    lax.fori_loop(0, RPS, zb, 0)

    pltpu.sync_copy(zbuf, s_sh.at[pl.ds(s * RPS, RPS)])
    pltpu.sync_copy(y_hbm.at[pl.ds(s * RPS, RPS)], y_sh.at[pl.ds(s * RPS, RPS)])
    plsc.subcore_barrier()

    bufs = (rows_a, rows_b)
    sems = (sem_a, sem_b)

    def chunk(g, carry):
        base = wid * RPT + g * 8
        pltpu.sync_copy(src_hbm.at[pl.ds(base, 8)], si_v)
        pltpu.sync_copy(dst_hbm.at[pl.ds(base, 8)], di_v)
        h = pltpu.async_copy(y_sh.at[si_v.at[0]], bufs[0], sems[0])
        for j in range(8):
            if j < 7:
                h_next = pltpu.async_copy(
                    y_sh.at[si_v.at[j + 1]], bufs[(j + 1) % 2], sems[(j + 1) % 2])
            h.wait()
            pltpu.sync_copy(bufs[j % 2], s_sh.at[di_v.at[j]], add=True)
            if j < 7:
                h = h_next
        return carry
    lax.fori_loop(0, RPT // 8, chunk, 0)

    plsc.subcore_barrier()

    @pl.when(s == 0)
    def _():
        pltpu.sync_copy(s_sh, out_hbm.at[c])


@functools.lru_cache(maxsize=None)
def _scat_call_cached():
    mesh = plsc.VectorSubcoreMesh(core_axis_name="c", subcore_axis_name="s")
    return pl.kernel(
        _scat_body,
        mesh=mesh,
        compiler_params=pltpu.CompilerParams(use_tc_tiling_on_sc=False),
        out_type=jax.ShapeDtypeStruct((2, NP, 48), jnp.float32),
        scratch_types=[
            pltpu.VMEM((8, 128), jnp.int32),
            pltpu.VMEM((8, 128), jnp.int32),
            pltpu.VMEM((128, 48), jnp.float32),
            pltpu.VMEM((128, 48), jnp.float32),
            pltpu.VMEM((RPS, 48), jnp.float32),
            pltpu.VMEM_SHARED((NP, 48), jnp.float32),
            pltpu.VMEM_SHARED((NP, 48), jnp.float32),
            pltpu.SemaphoreType.DMA,
            pltpu.SemaphoreType.DMA,
        ],
    )


def _scat_call(src_m, dst_m, y):
    return _scat_call_cached()(src_m, dst_m, y)


def _prep_body(x_ref, deg_ref, y_ref, dis_ref):
    deg = deg_ref[0, :, 0:1] + deg_ref[1, :, 0:1] + 1.0
    dis = lax.rsqrt(deg)
    xT = jnp.transpose(x_ref[...])
    y_ref[...] = jnp.concatenate(
        [xT * dis[:N], jnp.zeros((NP - N, 48), jnp.float32)], axis=0)
    dis_ref[...] = dis


def _prep_call(x2d, deg_part):
    return pl.pallas_call(
        _prep_body,
        out_shape=[
            jax.ShapeDtypeStruct((NP, 48), jnp.float32),
            jax.ShapeDtypeStruct((NP, 1), jnp.float32),
        ],
    )(x2d, deg_part)


def _fused_body(S_ref, y_ref, dis_ref, gcnW_ref, gcnb_row_ref, saW_ref,
                taW_ref, gcnb_col_ref, Wih_ref, Whh_ref, bih_ref, bhh_ref,
                projW_ref, projb_ref, out_ref):
    dcol = dis_ref[...]
    rowm = (lax.broadcasted_iota(jnp.int32, (NP, 1), 0) < N
            ).astype(jnp.float32)
    s = dcol * (S_ref[0] + S_ref[1] + y_ref[...])
    alpha = jnp.dot(gcnW_ref[...], saW_ref[...])
    beta = jnp.dot(gcnW_ref[...], taW_ref[...])
    gamma = jnp.dot(gcnb_row_ref[...], taW_ref[...])
    e1 = jnp.exp(alpha * s) * rowm
    sw = e1 / jnp.sum(e1, axis=0, keepdims=True)
    u = sw * s
    v = sw
    e2 = jnp.exp(beta * u + gamma * v) * rowm
    gi = lax.broadcasted_iota(jnp.int32, (BT, BT), 0) // TT
    gj = lax.broadcasted_iota(jnp.int32, (BT, BT), 1) // TT
    G = (gi == gj).astype(jnp.float32)
    D2 = jnp.dot(e2, G)
    tw = e2 / D2
    pT = jnp.transpose(tw * u)
    qT = jnp.transpose(tw * v)
    Aw = lax.dot_general(Wih_ref[...], gcnW_ref[...],
                         (((1,), (1,)), ((), ())))
    Ab = lax.dot_general(Wih_ref[...], gcnb_col_ref[...],
                         (((1,), (0,)), ((), ())))
    bi = bih_ref[...]
    bh = bhh_ref[...]
    Whh = Whh_ref[...]
    for b in range(BB):
        H = jnp.zeros((HID, N), jnp.float32)
        for t in range(TT):
            pt = pT[b * TT + t:b * TT + t + 1, :N]
            qt = qT[b * TT + t:b * TT + t + 1, :N]
            GI = Aw * pt + Ab * qt + bi
            GH = lax.dot_general(Whh, H,
                                 (((1,), (0,)), ((), ()))) + bh
            r = jax.nn.sigmoid(GI[0:HID] + GH[0:HID])
            z = jax.nn.sigmoid(GI[HID:2 * HID] + GH[HID:2 * HID])
            nn_ = jnp.tanh(GI[2 * HID:3 * HID] + r * GH[2 * HID:3 * HID])
            H = (1.0 - z) * nn_ + z * H
        OUT = lax.dot_general(projW_ref[...], H,
                              (((0,), (0,)), ((), ()))) + projb_ref[...]
        out_ref[b] = OUT


def _fused_call(S_part, y, dis_col, gcn_W, gcn_b_row, sa_W, ta_W,
                gcn_b_col, W_ih, W_hh, b_ih_col, b_hh_col,
                proj_W, proj_b_col):
    return pl.pallas_call(
        _fused_body,
        compiler_params=pltpu.CompilerParams(
            vmem_limit_bytes=110 * 1024 * 1024),
        out_shape=jax.ShapeDtypeStruct((BB, TT, N), jnp.float32),
    )(S_part, y, dis_col, gcn_W, gcn_b_row, sa_W, ta_W,
      gcn_b_col, W_ih, W_hh, b_ih_col, b_hh_col, proj_W, proj_b_col)


def kernel(x, edge_index, gcn_W, gcn_b, sa_W, sa_b, ta_W, ta_b,
           W_ih, W_hh, b_ih, b_hh, proj_W, proj_b):
    x2d = x.reshape(BT, N)
    src = edge_index[0].astype(jnp.int32)
    dst = edge_index[1].astype(jnp.int32)
    pad = jnp.full((EP - E,), N, jnp.int32)
    src_m = jnp.concatenate([src, pad]).reshape(ROWS, 128)
    dst_m = jnp.concatenate([dst, pad]).reshape(ROWS, 128)
    deg_part = _deg_call(dst_m)
    y, dis_col = _prep_call(x2d, deg_part)
    S_part = _scat_call(src_m, dst_m, y)

    out3 = _fused_call(S_part, y, dis_col, gcn_W, gcn_b.reshape(1, HID),
                       sa_W, ta_W, gcn_b.reshape(HID, 1), W_ih, W_hh,
                       b_ih.reshape(3 * HID, 1), b_hh.reshape(3 * HID, 1),
                       proj_W, proj_b.reshape(TT, 1))
    return out3[..., None]
